# baseline (speedup 1.0000x reference)
import functools

import jax
import jax.numpy as jnp
from jax import lax
from jax.experimental import pallas as pl
from jax.experimental.pallas import tpu as pltpu
from jax.experimental.pallas import tpu_sc as plsc

N = 10000
D = 128
E = 320000
NC = 2
NS = 16
NW = NC * NS
B = 64
TPW = 10240
EP = NW * TPW
NB = TPW // B
NP = 10240
RPT = NP // NS
NEG = -1.0e30
NBT = 2 * NB
NB0 = 176
NB1 = NBT - NB0


def _phase_a_body(x_r, wp_r, bp_r, wg_r, asv_r, adv_r,
                  z_r, as_r, ad_r, cb_r, mx_r):
    i = pl.program_id(0)
    xv = x_r[...]
    h = lax.dot_general(xv, wp_r[...], (((1,), (1,)), ((), ())),
                        preferred_element_type=jnp.float32) + bp_r[...]
    h = 0.5 * h * (1.0 + lax.erf(h * 0.7071067811865476))
    zv = lax.dot_general(h, wg_r[...], (((1,), (1,)), ((), ())),
                         preferred_element_type=jnp.float32)
    z_r[...] = zv
    asb = jnp.sum(zv * asv_r[...], axis=1, keepdims=True)
    adb = jnp.sum(zv * adv_r[...], axis=1, keepdims=True)
    as_r[...] = asb
    ad_r[...] = adb
    ms = jnp.max(asb)
    md = jnp.max(adb)

    @pl.when(i == 0)
    def _():
        mx_r[0] = ms
        mx_r[1] = md

    @pl.when(i > 0)
    def _():
        mx_r[0] = jnp.maximum(mx_r[0], ms)
        mx_r[1] = jnp.maximum(mx_r[1], md)

    @pl.when(i == pl.num_programs(0) - 1)
    def _():
        m = mx_r[0] + mx_r[1]
        cval = jnp.where(m >= 0.0, m, 0.2 * m)
        cb_r[...] = jnp.full((8, 128), cval, jnp.float32)


_phase_a = pl.pallas_call(
    _phase_a_body,
    grid=(5,),
    in_specs=[
        pl.BlockSpec((2000, 128), lambda i: (i, 0)),
        pl.BlockSpec((128, 128), lambda i: (0, 0)),
        pl.BlockSpec((1, 128), lambda i: (0, 0)),
        pl.BlockSpec((128, 128), lambda i: (0, 0)),
        pl.BlockSpec((1, 128), lambda i: (0, 0)),
        pl.BlockSpec((1, 128), lambda i: (0, 0)),
    ],
    out_specs=[
        pl.BlockSpec((2000, 128), lambda i: (i, 0)),
        pl.BlockSpec((2000, 1), lambda i: (i, 0)),
        pl.BlockSpec((2000, 1), lambda i: (i, 0)),
        pl.BlockSpec((8, 128), lambda i: (0, 0)),
    ],
    out_shape=[
        jax.ShapeDtypeStruct((N, D), jnp.float32),
        jax.ShapeDtypeStruct((N, 1), jnp.float32),
        jax.ShapeDtypeStruct((N, 1), jnp.float32),
        jax.ShapeDtypeStruct((8, 128), jnp.float32),
    ],
    scratch_shapes=[pltpu.SMEM((2,), jnp.float32)],
)


def _edge_body(ids, asl, adl, zp, cvec, z2,
               num_out, den_out,
               as_loc, ad_loc, den_loc, idsb0, idsb1, dstb0, dstb1, fb,
               rows0, rows1, cloc,
               num_sh, asp, adp, sem0, sem1, ssem0, ssem1):
    c = lax.axis_index("c")
    s = lax.axis_index("s")
    w = s * NC + c
    nbc = jnp.where(c == 0, NB0, NB1)
    tbase = jnp.where(c == 0, s * NB0, NS * NB0 + s * NB1)
    @pl.when(s == 0)
    def _():
        pltpu.sync_copy(asl, asp)
        pltpu.sync_copy(adl, adp)

    pltpu.sync_copy(cvec, cloc)
    pltpu.sync_copy(z2, num_sh.at[pl.ds(s * RPT, RPT)])

    z16 = jnp.zeros((16,), jnp.float32)

    def zero(j, c2):
        den_loc[pl.ds(j * 16, 16)] = z16
        return c2

    lax.fori_loop(0, N // 16, zero, 0)
    plsc.subcore_barrier()
    pltpu.sync_copy(asp, as_loc)
    pltpu.sync_copy(adp, ad_loc)
    cx = cloc[...]

    def fetch(g, idsb, rows, sem):
        bid = tbase + g
        pltpu.sync_copy(ids.at[pl.ds(bid * 2 * B, 2 * B)], idsb)
        pltpu.async_copy(zp.at[idsb.at[pl.ds(0, B)]], rows, sem)

    def do_block(g, idsb, rows, dstb, sem, ssem):
        pltpu.make_async_copy(zp.at[idsb.at[pl.ds(0, B)]], rows, sem).wait()
        for j in range(B // 16):
            sv = idsb[pl.ds(j * 16, 16)]
            dv = idsb[pl.ds(B + j * 16, 16)]
            dstb[pl.ds(j * 16, 16)] = dv
            a = plsc.load_gather(as_loc, [sv]) + plsc.load_gather(ad_loc, [dv])
            e = jnp.where(a >= 0.0, a, 0.2 * a)
            f = jnp.exp(e - cx)
            fb[pl.ds(j * 16, 16)] = f
            plsc.addupdate_scatter(den_loc, [dv], f)

        for j in range(B // 16):
            fB = fb[pl.ds(j * 16, 16)]
            for jj in range(16):
                fs = fB.at[jnp.full((16,), jj, jnp.int32)].get(
                    mode="promise_in_bounds")
                for r in range(D // 16):
                    sl = pl.ds(r * 16, 16)
                    rows[j * 16 + jj, sl] = rows[j * 16 + jj, sl] * fs

        pltpu.async_copy(rows, num_sh.at[dstb], ssem, add=True)

    def swait(rows, dstb, ssem):
        pltpu.make_async_copy(rows, num_sh.at[dstb], ssem).wait()

    fetch(0, idsb0, rows0, sem0)
    fetch(1, idsb1, rows1, sem1)

    def pair(k, carry):
        g0 = 2 * k
        do_block(g0, idsb0, rows0, dstb0, sem0, ssem0)

        @pl.when(g0 + 2 < nbc)
        def _():
            swait(rows0, dstb0, ssem0)
            fetch(g0 + 2, idsb0, rows0, sem0)

        do_block(g0 + 1, idsb1, rows1, dstb1, sem1, ssem1)

        @pl.when(g0 + 3 < nbc)
        def _():
            swait(rows1, dstb1, ssem1)
            fetch(g0 + 3, idsb1, rows1, sem1)

        return carry

    lax.fori_loop(0, nbc // 2, pair, 0)
    swait(rows0, dstb0, ssem0)
    swait(rows1, dstb1, ssem1)
    plsc.subcore_barrier()
    pltpu.sync_copy(num_sh.at[pl.ds(s * RPT, RPT)],
                    num_out.at[c, pl.ds(s * RPT, RPT)])
    pltpu.sync_copy(den_loc, den_out.at[w])


_edge_phase = pl.kernel(
    _edge_body,
    out_type=(
        jax.ShapeDtypeStruct((NC, NP, D), jnp.float32),
        jax.ShapeDtypeStruct((NW, N), jnp.float32),
    ),
    mesh=plsc.VectorSubcoreMesh(core_axis_name="c", subcore_axis_name="s",
                                num_cores=NC, num_subcores=NS),
    compiler_params=pltpu.CompilerParams(needs_layout_passes=False),
    scratch_types=[
        pltpu.VMEM((N + 16,), jnp.float32),
        pltpu.VMEM((N,), jnp.float32),
        pltpu.VMEM((N,), jnp.float32),
        pltpu.VMEM((2 * B,), jnp.int32),
        pltpu.VMEM((2 * B,), jnp.int32),
        pltpu.VMEM((B,), jnp.int32),
        pltpu.VMEM((B,), jnp.int32),
        pltpu.VMEM((B,), jnp.float32),
        pltpu.VMEM((B, D), jnp.float32),
        pltpu.VMEM((B, D), jnp.float32),
        pltpu.VMEM((16,), jnp.float32),
        pltpu.VMEM_SHARED((NP, D), jnp.float32),
        pltpu.VMEM_SHARED((N + 16,), jnp.float32),
        pltpu.VMEM_SHARED((N,), jnp.float32),
        pltpu.SemaphoreType.DMA,
        pltpu.SemaphoreType.DMA,
        pltpu.SemaphoreType.DMA,
        pltpu.SemaphoreType.DMA,
    ],
)


def _phase_c_body(num_r, den_r, bg_r, lg_r, lb_r, o_r):
    nv = num_r[...]
    sv = nv[0] + nv[1]
    dv = jnp.sum(den_r[...], axis=1, keepdims=True)
    r = jnp.where(dv > 0.0, sv / dv, 0.0) + bg_r[...]
    mu = jnp.mean(r, axis=1, keepdims=True)
    var = jnp.mean((r - mu) ** 2, axis=1, keepdims=True)
    o_r[...] = (r - mu) * lax.rsqrt(var + 1e-5) * lg_r[...] + lb_r[...]


_phase_c = pl.pallas_call(
    _phase_c_body,
    grid=(5,),
    in_specs=[
        pl.BlockSpec((2, 2000, 128), lambda i: (0, i, 0)),
        pl.BlockSpec((2000, 32), lambda i: (i, 0)),
        pl.BlockSpec((1, 128), lambda i: (0, 0)),
        pl.BlockSpec((1, 128), lambda i: (0, 0)),
        pl.BlockSpec((1, 128), lambda i: (0, 0)),
    ],
    out_specs=pl.BlockSpec((2000, 128), lambda i: (i, 0)),
    out_shape=jax.ShapeDtypeStruct((N, D), jnp.float32),
)


def kernel(x, edge_index, W_proj, b_proj, W_gat, att_src, att_dst, b_gat,
           ln_g, ln_b):
    z, as1, ad1, cb = _phase_a(
        x, W_proj, b_proj.reshape(1, D), W_gat,
        att_src.reshape(1, D), att_dst.reshape(1, D))

    src = edge_index[0]
    dst = edge_index[1]
    pad = EP - E
    srcp = jnp.concatenate([src, jnp.full((pad,), N, jnp.int32)])
    dstp = jnp.concatenate([dst, jnp.zeros((pad,), jnp.int32)])
    ids = jnp.stack([srcp.reshape(EP // B, B),
                     dstp.reshape(EP // B, B)], axis=1).reshape(2 * EP)
    asl = jnp.concatenate([as1[:, 0], jnp.full((16,), NEG, jnp.float32)])
    adl = ad1[:, 0]
    zpad = jnp.concatenate([z, jnp.zeros((16, D), jnp.float32)], axis=0)
    cvec = jnp.full((16,), cb[0, 0], jnp.float32)
    z2 = jnp.zeros((RPT, D), jnp.float32)

    num, den = _edge_phase(ids, asl, adl, zpad, cvec, z2)

    out = _phase_c(num[:, :N], den.T, b_gat.reshape(1, D), ln_g.reshape(1, D),
                   ln_b.reshape(1, D))
    return out


# speedup vs baseline: 1.0574x; 1.0574x over previous
import functools

import jax
import jax.numpy as jnp
from jax import lax
from jax.experimental import pallas as pl
from jax.experimental.pallas import tpu as pltpu
from jax.experimental.pallas import tpu_sc as plsc

N = 10000
D = 128
E = 320000
NC = 2
NS = 16
NW = NC * NS
B = 64
TPW = 10240
EP = NW * TPW
NB = TPW // B
NP = 10240
RPT = NP // NS
NEG = -1.0e30
NBT = 2 * NB
NB0 = 200
NB1 = NBT - NB0


def _phase_a_body(x_r, wp_r, bp_r, wg_r, asv_r, adv_r,
                  z_r, as_r, ad_r, cb_r, mx_r):
    i = pl.program_id(0)
    xv = x_r[...]
    h = lax.dot_general(xv, wp_r[...], (((1,), (1,)), ((), ())),
                        preferred_element_type=jnp.float32) + bp_r[...]
    h = 0.5 * h * (1.0 + lax.erf(h * 0.7071067811865476))
    zv = lax.dot_general(h, wg_r[...], (((1,), (1,)), ((), ())),
                         preferred_element_type=jnp.float32)
    z_r[...] = zv
    asb = jnp.sum(zv * asv_r[...], axis=1, keepdims=True)
    adb = jnp.sum(zv * adv_r[...], axis=1, keepdims=True)
    as_r[...] = asb
    ad_r[...] = adb
    ms = jnp.max(asb)
    md = jnp.max(adb)

    @pl.when(i == 0)
    def _():
        mx_r[0] = ms
        mx_r[1] = md

    @pl.when(i > 0)
    def _():
        mx_r[0] = jnp.maximum(mx_r[0], ms)
        mx_r[1] = jnp.maximum(mx_r[1], md)

    @pl.when(i == pl.num_programs(0) - 1)
    def _():
        m = mx_r[0] + mx_r[1]
        cval = jnp.where(m >= 0.0, m, 0.2 * m)
        cb_r[...] = jnp.full((8, 128), cval, jnp.float32)


_phase_a = pl.pallas_call(
    _phase_a_body,
    grid=(5,),
    in_specs=[
        pl.BlockSpec((2000, 128), lambda i: (i, 0)),
        pl.BlockSpec((128, 128), lambda i: (0, 0)),
        pl.BlockSpec((1, 128), lambda i: (0, 0)),
        pl.BlockSpec((128, 128), lambda i: (0, 0)),
        pl.BlockSpec((1, 128), lambda i: (0, 0)),
        pl.BlockSpec((1, 128), lambda i: (0, 0)),
    ],
    out_specs=[
        pl.BlockSpec((2000, 128), lambda i: (i, 0)),
        pl.BlockSpec((2000, 1), lambda i: (i, 0)),
        pl.BlockSpec((2000, 1), lambda i: (i, 0)),
        pl.BlockSpec((8, 128), lambda i: (0, 0)),
    ],
    out_shape=[
        jax.ShapeDtypeStruct((N, D), jnp.float32),
        jax.ShapeDtypeStruct((N, 1), jnp.float32),
        jax.ShapeDtypeStruct((N, 1), jnp.float32),
        jax.ShapeDtypeStruct((8, 128), jnp.float32),
    ],
    scratch_shapes=[pltpu.SMEM((2,), jnp.float32)],
)


def _edge_body(ids, asl, adl, zp, cvec, z2,
               num_out, den_out,
               as_loc, ad_loc, den_loc, idsb0, idsb1, dstb0, dstb1, fb,
               rows0, rows1, cloc,
               num_sh, asp, adp, sem0, sem1, ssem0, ssem1):
    c = lax.axis_index("c")
    s = lax.axis_index("s")
    w = s * NC + c
    nbc = jnp.where(c == 0, NB0, NB1)
    tbase = jnp.where(c == 0, s * NB0, NS * NB0 + s * NB1)
    @pl.when(s == 0)
    def _():
        pltpu.sync_copy(asl, asp)
        pltpu.sync_copy(adl, adp)

    pltpu.sync_copy(cvec, cloc)
    pltpu.sync_copy(z2, num_sh.at[pl.ds(s * RPT, RPT)])

    z16 = jnp.zeros((16,), jnp.float32)

    def zero(j, c2):
        den_loc[pl.ds(j * 16, 16)] = z16
        return c2

    lax.fori_loop(0, N // 16, zero, 0)
    plsc.subcore_barrier()
    pltpu.sync_copy(asp, as_loc)
    pltpu.sync_copy(adp, ad_loc)
    cx = cloc[...]

    def fetch(g, idsb, rows, sem):
        bid = tbase + g
        pltpu.sync_copy(ids.at[pl.ds(bid * 2 * B, 2 * B)], idsb)
        pltpu.async_copy(zp.at[idsb.at[pl.ds(0, B)]], rows, sem)

    def do_block(g, idsb, rows, dstb, sem, ssem):
        pltpu.make_async_copy(zp.at[idsb.at[pl.ds(0, B)]], rows, sem).wait()
        for j in range(B // 16):
            sv = idsb[pl.ds(j * 16, 16)]
            dv = idsb[pl.ds(B + j * 16, 16)]
            dstb[pl.ds(j * 16, 16)] = dv
            a = plsc.load_gather(as_loc, [sv]) + plsc.load_gather(ad_loc, [dv])
            e = jnp.where(a >= 0.0, a, 0.2 * a)
            f = jnp.exp(e - cx)
            fb[pl.ds(j * 16, 16)] = f
            plsc.addupdate_scatter(den_loc, [dv], f)

        for j in range(B // 16):
            fB = fb[pl.ds(j * 16, 16)]
            for jj in range(16):
                fs = fB.at[jnp.full((16,), jj, jnp.int32)].get(
                    mode="promise_in_bounds")
                for r in range(D // 16):
                    sl = pl.ds(r * 16, 16)
                    rows[j * 16 + jj, sl] = rows[j * 16 + jj, sl] * fs

        pltpu.async_copy(rows, num_sh.at[dstb], ssem, add=True)

    def swait(rows, dstb, ssem):
        pltpu.make_async_copy(rows, num_sh.at[dstb], ssem).wait()

    fetch(0, idsb0, rows0, sem0)
    fetch(1, idsb1, rows1, sem1)

    def pair(k, carry):
        g0 = 2 * k
        do_block(g0, idsb0, rows0, dstb0, sem0, ssem0)

        @pl.when(g0 + 2 < nbc)
        def _():
            swait(rows0, dstb0, ssem0)
            fetch(g0 + 2, idsb0, rows0, sem0)

        do_block(g0 + 1, idsb1, rows1, dstb1, sem1, ssem1)

        @pl.when(g0 + 3 < nbc)
        def _():
            swait(rows1, dstb1, ssem1)
            fetch(g0 + 3, idsb1, rows1, sem1)

        return carry

    lax.fori_loop(0, nbc // 2, pair, 0)
    swait(rows0, dstb0, ssem0)
    swait(rows1, dstb1, ssem1)
    plsc.subcore_barrier()
    pltpu.sync_copy(num_sh.at[pl.ds(s * RPT, RPT)],
                    num_out.at[c, pl.ds(s * RPT, RPT)])
    pltpu.sync_copy(den_loc, den_out.at[w])


_edge_phase = pl.kernel(
    _edge_body,
    out_type=(
        jax.ShapeDtypeStruct((NC, NP, D), jnp.float32),
        jax.ShapeDtypeStruct((NW, N), jnp.float32),
    ),
    mesh=plsc.VectorSubcoreMesh(core_axis_name="c", subcore_axis_name="s",
                                num_cores=NC, num_subcores=NS),
    compiler_params=pltpu.CompilerParams(needs_layout_passes=False),
    scratch_types=[
        pltpu.VMEM((N + 16,), jnp.float32),
        pltpu.VMEM((N,), jnp.float32),
        pltpu.VMEM((N,), jnp.float32),
        pltpu.VMEM((2 * B,), jnp.int32),
        pltpu.VMEM((2 * B,), jnp.int32),
        pltpu.VMEM((B,), jnp.int32),
        pltpu.VMEM((B,), jnp.int32),
        pltpu.VMEM((B,), jnp.float32),
        pltpu.VMEM((B, D), jnp.float32),
        pltpu.VMEM((B, D), jnp.float32),
        pltpu.VMEM((16,), jnp.float32),
        pltpu.VMEM_SHARED((NP, D), jnp.float32),
        pltpu.VMEM_SHARED((N + 16,), jnp.float32),
        pltpu.VMEM_SHARED((N,), jnp.float32),
        pltpu.SemaphoreType.DMA,
        pltpu.SemaphoreType.DMA,
        pltpu.SemaphoreType.DMA,
        pltpu.SemaphoreType.DMA,
    ],
)


def _phase_c_body(num_r, den_r, bg_r, lg_r, lb_r, o_r):
    nv = num_r[...]
    sv = nv[0] + nv[1]
    dv = jnp.sum(den_r[...], axis=1, keepdims=True)
    r = jnp.where(dv > 0.0, sv / dv, 0.0) + bg_r[...]
    mu = jnp.mean(r, axis=1, keepdims=True)
    var = jnp.mean((r - mu) ** 2, axis=1, keepdims=True)
    o_r[...] = (r - mu) * lax.rsqrt(var + 1e-5) * lg_r[...] + lb_r[...]


_phase_c = pl.pallas_call(
    _phase_c_body,
    grid=(5,),
    in_specs=[
        pl.BlockSpec((2, 2000, 128), lambda i: (0, i, 0)),
        pl.BlockSpec((2000, 32), lambda i: (i, 0)),
        pl.BlockSpec((1, 128), lambda i: (0, 0)),
        pl.BlockSpec((1, 128), lambda i: (0, 0)),
        pl.BlockSpec((1, 128), lambda i: (0, 0)),
    ],
    out_specs=pl.BlockSpec((2000, 128), lambda i: (i, 0)),
    out_shape=jax.ShapeDtypeStruct((N, D), jnp.float32),
)


def kernel(x, edge_index, W_proj, b_proj, W_gat, att_src, att_dst, b_gat,
           ln_g, ln_b):
    z, as1, ad1, cb = _phase_a(
        x, W_proj, b_proj.reshape(1, D), W_gat,
        att_src.reshape(1, D), att_dst.reshape(1, D))

    src = edge_index[0]
    dst = edge_index[1]
    pad = EP - E
    srcp = jnp.concatenate([src, jnp.full((pad,), N, jnp.int32)])
    dstp = jnp.concatenate([dst, jnp.zeros((pad,), jnp.int32)])
    ids = jnp.stack([srcp.reshape(EP // B, B),
                     dstp.reshape(EP // B, B)], axis=1).reshape(2 * EP)
    asl = jnp.concatenate([as1[:, 0], jnp.full((16,), NEG, jnp.float32)])
    adl = ad1[:, 0]
    zpad = jnp.concatenate([z, jnp.zeros((16, D), jnp.float32)], axis=0)
    cvec = jnp.full((16,), cb[0, 0], jnp.float32)
    z2 = jnp.zeros((RPT, D), jnp.float32)

    num, den = _edge_phase(ids, asl, adl, zpad, cvec, z2)

    out = _phase_c(num[:, :N], den.T, b_gat.reshape(1, D), ln_g.reshape(1, D),
                   ln_b.reshape(1, D))
    return out


# speedup vs baseline: 1.2162x; 1.1502x over previous
import functools

import jax
import jax.numpy as jnp
from jax import lax
from jax.experimental import pallas as pl
from jax.experimental.pallas import tpu as pltpu
from jax.experimental.pallas import tpu_sc as plsc

N = 10000
D = 128
E = 320000
NC = 2
NS = 16
NW = NC * NS
B = 48
NBT = 420
EP = NS * NBT * B
NP = 10112
RPT = NP // NS
NEG = -1.0e30
NB0 = 262
NB1 = NBT - NB0


def _phase_a_body(x_r, wp_r, bp_r, wg_r, asv_r, adv_r,
                  z_r, as_r, ad_r, cb_r, mx_r):
    i = pl.program_id(0)
    xv = x_r[...]
    h = lax.dot_general(xv, wp_r[...], (((1,), (1,)), ((), ())),
                        preferred_element_type=jnp.float32) + bp_r[...]
    h = 0.5 * h * (1.0 + lax.erf(h * 0.7071067811865476))
    zv = lax.dot_general(h, wg_r[...], (((1,), (1,)), ((), ())),
                         preferred_element_type=jnp.float32)
    z_r[...] = zv
    asb = jnp.sum(zv * asv_r[...], axis=1, keepdims=True)
    adb = jnp.sum(zv * adv_r[...], axis=1, keepdims=True)
    as_r[...] = asb
    ad_r[...] = adb
    ms = jnp.max(asb)
    md = jnp.max(adb)

    @pl.when(i == 0)
    def _():
        mx_r[0] = ms
        mx_r[1] = md

    @pl.when(i > 0)
    def _():
        mx_r[0] = jnp.maximum(mx_r[0], ms)
        mx_r[1] = jnp.maximum(mx_r[1], md)

    @pl.when(i == pl.num_programs(0) - 1)
    def _():
        m = mx_r[0] + mx_r[1]
        cval = jnp.where(m >= 0.0, m, 0.2 * m)
        cb_r[...] = jnp.full((8, 128), cval, jnp.float32)


_phase_a = pl.pallas_call(
    _phase_a_body,
    grid=(5,),
    in_specs=[
        pl.BlockSpec((2000, 128), lambda i: (i, 0)),
        pl.BlockSpec((128, 128), lambda i: (0, 0)),
        pl.BlockSpec((1, 128), lambda i: (0, 0)),
        pl.BlockSpec((128, 128), lambda i: (0, 0)),
        pl.BlockSpec((1, 128), lambda i: (0, 0)),
        pl.BlockSpec((1, 128), lambda i: (0, 0)),
    ],
    out_specs=[
        pl.BlockSpec((2000, 128), lambda i: (i, 0)),
        pl.BlockSpec((2000, 1), lambda i: (i, 0)),
        pl.BlockSpec((2000, 1), lambda i: (i, 0)),
        pl.BlockSpec((8, 128), lambda i: (0, 0)),
    ],
    out_shape=[
        jax.ShapeDtypeStruct((N, D), jnp.float32),
        jax.ShapeDtypeStruct((N, 1), jnp.float32),
        jax.ShapeDtypeStruct((N, 1), jnp.float32),
        jax.ShapeDtypeStruct((8, 128), jnp.float32),
    ],
    scratch_shapes=[pltpu.SMEM((2,), jnp.float32)],
)


def _edge_body(ids, asl, adl, zp, cvec, z2,
               num_out, den_out,
               as_loc, ad_loc, den_loc, idsb0, idsb1, idsb2,
               dstb0, dstb1, dstb2, fb,
               rows0, rows1, rows2, cloc,
               num_sh, sem0, sem1, sem2, ssem0, ssem1, ssem2):
    c = lax.axis_index("c")
    s = lax.axis_index("s")
    w = s * NC + c
    nbc = jnp.where(c == 0, NB0, NB1)
    tbase = jnp.where(c == 0, s * NB0, NS * NB0 + s * NB1)
    pltpu.sync_copy(asl, as_loc)
    pltpu.sync_copy(adl, ad_loc)
    pltpu.sync_copy(cvec, cloc)
    pltpu.sync_copy(z2, num_sh.at[pl.ds(s * RPT, RPT)])

    z16 = jnp.zeros((16,), jnp.float32)

    def zero(j, c2):
        den_loc[pl.ds(j * 16, 16)] = z16
        return c2

    lax.fori_loop(0, N // 16, zero, 0)
    plsc.subcore_barrier()
    cx = cloc[...]

    def fetch(g, idsb, rows, sem):
        bid = tbase + g
        pltpu.sync_copy(ids.at[pl.ds(bid * 2 * B, 2 * B)], idsb)
        pltpu.async_copy(zp.at[idsb.at[pl.ds(0, B)]], rows, sem)

    def do_block(g, idsb, rows, dstb, sem, ssem):
        pltpu.make_async_copy(zp.at[idsb.at[pl.ds(0, B)]], rows, sem).wait()
        for j in range(B // 16):
            sv = idsb[pl.ds(j * 16, 16)]
            dv = idsb[pl.ds(B + j * 16, 16)]
            dstb[pl.ds(j * 16, 16)] = dv
            a = plsc.load_gather(as_loc, [sv]) + plsc.load_gather(ad_loc, [dv])
            e = jnp.where(a >= 0.0, a, 0.2 * a)
            f = jnp.exp(e - cx)
            fb[pl.ds(j * 16, 16)] = f
            plsc.addupdate_scatter(den_loc, [dv], f)

        for j in range(B // 16):
            fB = fb[pl.ds(j * 16, 16)]
            for jj in range(16):
                fs = fB.at[jnp.full((16,), jj, jnp.int32)].get(
                    mode="promise_in_bounds")
                for r in range(D // 16):
                    sl = pl.ds(r * 16, 16)
                    rows[j * 16 + jj, sl] = rows[j * 16 + jj, sl] * fs

        pltpu.async_copy(rows, num_sh.at[dstb], ssem, add=True)

    def swait(rows, dstb, ssem):
        pltpu.make_async_copy(rows, num_sh.at[dstb], ssem).wait()

    fetch(0, idsb0, rows0, sem0)
    fetch(1, idsb1, rows1, sem1)
    fetch(2, idsb2, rows2, sem2)
    slots = ((idsb0, rows0, dstb0, sem0, ssem0),
             (idsb1, rows1, dstb1, sem1, ssem1),
             (idsb2, rows2, dstb2, sem2, ssem2))

    def trio(k, carry):
        g0 = 3 * k
        for i in range(3):
            idsb, rows, dstb, sem, ssem = slots[i]
            g = g0 + i

            @pl.when(g < nbc)
            def _():
                do_block(g, idsb, rows, dstb, sem, ssem)

            @pl.when(g + 3 < nbc)
            def _():
                swait(rows, dstb, ssem)
                fetch(g + 3, idsb, rows, sem)

        return carry

    lax.fori_loop(0, (nbc + 2) // 3, trio, 0)
    for i in range(3):
        idsb, rows, dstb, sem, ssem = slots[i]
        swait(rows, dstb, ssem)
    plsc.subcore_barrier()
    pltpu.sync_copy(num_sh.at[pl.ds(s * RPT, RPT)],
                    num_out.at[c, pl.ds(s * RPT, RPT)])
    pltpu.sync_copy(den_loc, den_out.at[w])


_edge_phase = pl.kernel(
    _edge_body,
    out_type=(
        jax.ShapeDtypeStruct((NC, NP, D), jnp.float32),
        jax.ShapeDtypeStruct((NW, N), jnp.float32),
    ),
    mesh=plsc.VectorSubcoreMesh(core_axis_name="c", subcore_axis_name="s",
                                num_cores=NC, num_subcores=NS),
    compiler_params=pltpu.CompilerParams(needs_layout_passes=False),
    scratch_types=[
        pltpu.VMEM((N + 16,), jnp.float32),
        pltpu.VMEM((N,), jnp.float32),
        pltpu.VMEM((N,), jnp.float32),
        pltpu.VMEM((2 * B,), jnp.int32),
        pltpu.VMEM((2 * B,), jnp.int32),
        pltpu.VMEM((2 * B,), jnp.int32),
        pltpu.VMEM((B,), jnp.int32),
        pltpu.VMEM((B,), jnp.int32),
        pltpu.VMEM((B,), jnp.int32),
        pltpu.VMEM((B,), jnp.float32),
        pltpu.VMEM((B, D), jnp.float32),
        pltpu.VMEM((B, D), jnp.float32),
        pltpu.VMEM((B, D), jnp.float32),
        pltpu.VMEM((16,), jnp.float32),
        pltpu.VMEM_SHARED((NP, D), jnp.float32),
        pltpu.SemaphoreType.DMA,
        pltpu.SemaphoreType.DMA,
        pltpu.SemaphoreType.DMA,
        pltpu.SemaphoreType.DMA,
        pltpu.SemaphoreType.DMA,
        pltpu.SemaphoreType.DMA,
    ],
)


def _phase_c_body(num_r, den_r, bg_r, lg_r, lb_r, o_r):
    nv = num_r[...]
    sv = nv[0] + nv[1]
    dv = jnp.sum(den_r[...], axis=1, keepdims=True)
    r = jnp.where(dv > 0.0, sv / dv, 0.0) + bg_r[...]
    mu = jnp.mean(r, axis=1, keepdims=True)
    var = jnp.mean((r - mu) ** 2, axis=1, keepdims=True)
    o_r[...] = (r - mu) * lax.rsqrt(var + 1e-5) * lg_r[...] + lb_r[...]


_phase_c = pl.pallas_call(
    _phase_c_body,
    grid=(5,),
    in_specs=[
        pl.BlockSpec((2, 2000, 128), lambda i: (0, i, 0)),
        pl.BlockSpec((2000, 32), lambda i: (i, 0)),
        pl.BlockSpec((1, 128), lambda i: (0, 0)),
        pl.BlockSpec((1, 128), lambda i: (0, 0)),
        pl.BlockSpec((1, 128), lambda i: (0, 0)),
    ],
    out_specs=pl.BlockSpec((2000, 128), lambda i: (i, 0)),
    out_shape=jax.ShapeDtypeStruct((N, D), jnp.float32),
)


def kernel(x, edge_index, W_proj, b_proj, W_gat, att_src, att_dst, b_gat,
           ln_g, ln_b):
    z, as1, ad1, cb = _phase_a(
        x, W_proj, b_proj.reshape(1, D), W_gat,
        att_src.reshape(1, D), att_dst.reshape(1, D))

    src = edge_index[0]
    dst = edge_index[1]
    pad = EP - E
    srcp = jnp.concatenate([src, jnp.full((pad,), N, jnp.int32)])
    dstp = jnp.concatenate([dst, jnp.zeros((pad,), jnp.int32)])
    ids = jnp.stack([srcp.reshape(EP // B, B),
                     dstp.reshape(EP // B, B)], axis=1).reshape(2 * EP)
    asl = jnp.concatenate([as1[:, 0], jnp.full((16,), NEG, jnp.float32)])
    adl = ad1[:, 0]
    zpad = jnp.concatenate([z, jnp.zeros((16, D), jnp.float32)], axis=0)
    cvec = jnp.full((16,), cb[0, 0], jnp.float32)
    z2 = jnp.zeros((RPT, D), jnp.float32)

    num, den = _edge_phase(ids, asl, adl, zpad, cvec, z2)

    out = _phase_c(num[:, :N], den.T, b_gat.reshape(1, D), ln_g.reshape(1, D),
                   ln_b.reshape(1, D))
    return out
